# baseline (speedup 1.0000x reference)
import jax
import jax.numpy as jnp
from jax import lax
from jax.experimental import pallas as pl
from jax.experimental.pallas import tpu as pltpu

_H = 512
_W = 512
_C = 192
_R = 8
_NCORE = 2
_STEPS = (_H // _NCORE) // _R

_mesh = pltpu.create_tensorcore_mesh("core", num_cores=_NCORE)


def _body(o_hbm, s_hbm, out_hbm, obuf, sbuf, rbuf, in_sems, out_sems):
    core = lax.axis_index("core")
    base = core * (_H // _NCORE)

    def in_copies(step, slot):
        r0 = base + step * _R
        return (
            pltpu.make_async_copy(
                o_hbm.at[0, pl.ds(r0, _R)], obuf.at[slot], in_sems.at[slot, 0]
            ),
            pltpu.make_async_copy(
                s_hbm.at[0, pl.ds(r0, _R)], sbuf.at[slot], in_sems.at[slot, 1]
            ),
        )

    def out_copy(step, slot):
        r0 = base + step * _R
        return pltpu.make_async_copy(
            rbuf.at[slot], out_hbm.at[0, pl.ds(r0, _R)], out_sems.at[slot]
        )

    for c in in_copies(0, 0):
        c.start()

    def loop(step, carry):
        slot = lax.rem(step, 2)
        nxt = 1 - slot

        @pl.when(step + 1 < _STEPS)
        def _():
            for c in in_copies(step + 1, nxt):
                c.start()

        for c in in_copies(step, slot):
            c.wait()

        o = obuf[slot]
        s = sbuf[slot]
        dot = jnp.sum(o * s, axis=2, keepdims=True)
        n1 = jnp.sqrt(jnp.sum(o * o, axis=2, keepdims=True))
        n2 = jnp.sqrt(jnp.sum(s * s, axis=2, keepdims=True))
        eps = jnp.float32(1e-8)
        scores = dot / (jnp.maximum(n1, eps) * jnp.maximum(n2, eps))
        row = base + step * _R + lax.broadcasted_iota(jnp.int32, (_R, _W, 1), 0)
        col = lax.broadcasted_iota(jnp.int32, (_R, _W, 1), 1)
        mask = (
            (scores > 0.17)
            & (scores < 0.29)
            & (row > 0)
            & (col > 0)
        )

        @pl.when(step >= 2)
        def _():
            out_copy(step - 2, slot).wait()

        rbuf[slot] = jnp.where(mask, s, o)
        out_copy(step, slot).start()
        return carry

    lax.fori_loop(0, _STEPS, loop, 0)
    out_copy(_STEPS - 2, lax.rem(_STEPS - 2, 2)).wait()
    out_copy(_STEPS - 1, lax.rem(_STEPS - 1, 2)).wait()


def kernel(original, styled):
    f = pl.kernel(
        _body,
        out_type=jax.ShapeDtypeStruct((1, _H, _W, _C), jnp.float32),
        mesh=_mesh,
        scratch_types=[
            pltpu.VMEM((2, _R, _W, _C), jnp.float32),
            pltpu.VMEM((2, _R, _W, _C), jnp.float32),
            pltpu.VMEM((2, _R, _W, _C), jnp.float32),
            pltpu.SemaphoreType.DMA((2, 2)),
            pltpu.SemaphoreType.DMA((2,)),
        ],
    )
    return f(original, styled)
